# baseline (speedup 1.0000x reference)
import functools
import math

import jax
import jax.numpy as jnp
from jax import lax
from jax.experimental import pallas as pl
from jax.experimental.pallas import tpu as pltpu
from jax.experimental.pallas import tpu_sc as plsc

_N_BOXES = 20000
_NUM_CLASSES = 80
_NUM_PREDS = 100
_IOU_THR = 0.5
_SCORE_THR = 0.3
_MAX_EDGE = 1024
_SCALE_CLAMP = math.log(1000.0 / 16)

_ROWS = 160
_LANES = 128
_NPAD = _ROWS * _LANES

_SC_TILES = 16
_SC_CHUNK = _NPAD // _SC_TILES
_SC_STEPS = _SC_CHUNK // 16


def _dense_body(l_ref, ax1_ref, ay1_ref, ax2_ref, ay2_ref,
                dx_ref, dy_ref, dw_ref, dh_ref, sc8_ref):
    f32 = jnp.float32

    ax1, ay1, ax2, ay2 = ax1_ref[...], ay1_ref[...], ax2_ref[...], ay2_ref[...]
    dxv, dyv, dwv, dhv = dx_ref[...], dy_ref[...], dw_ref[...], dh_ref[...]
    widths = ax2 - ax1
    heights = ay2 - ay1
    ctr_x = (ax1 + ax2) * 0.5
    ctr_y = (ay1 + ay2) * 0.5
    dw = jnp.minimum(dwv, _SCALE_CLAMP)
    dh = jnp.minimum(dhv, _SCALE_CLAMP)
    pred_ctr_x = dxv * widths + ctr_x
    pred_ctr_y = dyv * heights + ctr_y
    pred_w = jnp.exp(dw) * widths
    pred_h = jnp.exp(dh) * heights
    hi = f32(_MAX_EDGE - 1.0)
    x1 = jnp.clip(pred_ctr_x - 0.5 * pred_w, 0.0, hi)
    y1 = jnp.clip(pred_ctr_y - 0.5 * pred_h, 0.0, hi)
    x2 = jnp.clip(pred_ctr_x + 0.5 * pred_w, 0.0, hi)
    y2 = jnp.clip(pred_ctr_y + 0.5 * pred_h, 0.0, hi)

    l = jnp.concatenate(
        [l_ref[...],
         jnp.full((_NPAD - _N_BOXES, _NUM_CLASSES), -100.0, jnp.float32)],
        axis=0).reshape(_ROWS, _LANES, _NUM_CLASSES)
    p = jax.nn.sigmoid(l)
    m = jnp.max(p, axis=2)
    cat = jnp.argmax(p, axis=2)
    catf = cat.astype(f32)
    s0 = jnp.where(m >= _SCORE_THR, m, -1.0)

    off = catf * f32(_MAX_EDGE)
    x1c = x1 + off
    y1c = y1 + off
    x2c = x2 + off
    y2c = y2 + off
    area = (x2c - x1c) * (y2c - y1c)

    sc8_ref[0] = s0
    sc8_ref[1] = x1c
    sc8_ref[2] = y1c
    sc8_ref[3] = x2c
    sc8_ref[4] = y2c
    sc8_ref[5] = area
    sc8_ref[6] = catf
    sc8_ref[7] = jnp.zeros((_ROWS, _LANES), f32)


def _allmax_f(v):
    iota = lax.iota(jnp.int32, 16)
    for k in (1, 2, 4, 8):
        v = jnp.maximum(v, jnp.take_along_axis(v, iota ^ k, axis=0))
    return v


def _allmin_f(v):
    iota = lax.iota(jnp.int32, 16)
    for k in (1, 2, 4, 8):
        v = jnp.minimum(v, jnp.take_along_axis(v, iota ^ k, axis=0))
    return v


def _allmin_i(v):
    iota = lax.iota(jnp.int32, 16)
    for k in (1, 2, 4, 8):
        v = jnp.minimum(v, jnp.take_along_axis(v, iota ^ k, axis=0))
    return v


def _nms_body(sc8_hbm, pred_hbm, loc, slot, rec, outbuf, shared):
    f32 = jnp.float32
    i32 = jnp.int32
    t = lax.axis_index("s")
    core = lax.axis_index("c")
    base = t * _SC_CHUNK
    iota16 = lax.iota(i32, 16)

    def bcl(v, k):
        return jnp.take_along_axis(v, iota16 * 0 + k, axis=0)

    for q in range(8):
        pltpu.sync_copy(sc8_hbm.at[q, t], loc.at[pl.ds(q * _SC_CHUNK, _SC_CHUNK)])

    def scan_pass(wx1, wy1, wx2, wy2, wa, jgv):
        def step(k, carry):
            bv, bi = carry
            o = k * 16
            sv = loc[pl.ds(o, 16)]
            xx1 = jnp.maximum(wx1, loc[pl.ds(1 * _SC_CHUNK + o, 16)])
            yy1 = jnp.maximum(wy1, loc[pl.ds(2 * _SC_CHUNK + o, 16)])
            xx2 = jnp.minimum(wx2, loc[pl.ds(3 * _SC_CHUNK + o, 16)])
            yy2 = jnp.minimum(wy2, loc[pl.ds(4 * _SC_CHUNK + o, 16)])
            inter = jnp.maximum(xx2 - xx1, 0.0) * jnp.maximum(yy2 - yy1, 0.0)
            iou = inter / (wa + loc[pl.ds(5 * _SC_CHUNK + o, 16)] - inter + 1e-9)
            gidx = iota16 + (base + o)
            supp = (iou > _IOU_THR) | (gidx == jgv)
            snew = jnp.where(supp, -1.0, sv)
            loc[pl.ds(o, 16)] = snew
            cond = snew > bv
            bv = jnp.where(cond, snew, bv)
            bi = jnp.where(cond, gidx, bi)
            return bv, bi
        init = (jnp.full((16,), -2.0, f32), jnp.zeros((16,), i32))
        return lax.fori_loop(0, _SC_STEPS, step, init)

    big = f32(3.0e38)
    bv0, bi0 = scan_pass(jnp.full((16,), big, f32), jnp.full((16,), big, f32),
                         jnp.full((16,), -big, f32), jnp.full((16,), -big, f32),
                         jnp.full((16,), 1.0, f32),
                         jnp.full((16,), -1, i32))

    def nms_iter(i, carry):
        bv, bi = carry
        mv = _allmax_f(bv)
        fidv = _allmin_i(jnp.where(bv == mv, bi, i32(2**30)))
        fid = fidv[0]
        jl = fid - base
        jb = (jl // 16) * 16
        ofsv = iota16 * 0 + (jl - jb)

        recv = jnp.full((16,), -1.0, f32)
        for q in range(7):
            wq = loc[pl.ds(q * _SC_CHUNK + jb, 16)]
            vq = jnp.take_along_axis(wq, ofsv, axis=0)
            recv = jnp.where(iota16 == q, vq, recv)
        recv = jnp.where(iota16 == 7, fidv.astype(f32), recv)

        off = (i % 2) * 192
        slot[...] = jnp.where(iota16 == 0, mv,
                    jnp.where(iota16 == 1, fidv.astype(f32), 0.0))
        pltpu.sync_copy(slot.at[pl.ds(0, 8)], shared.at[pl.ds(off + 8 * t, 8)])
        plsc.subcore_barrier()

        s16 = jnp.full((16,), -3.0, f32)
        f16 = jnp.full((16,), 0.0, f32)
        for k in range(8):
            pltpu.sync_copy(shared.at[pl.ds(off + 16 * k, 16)], slot)
            vk = slot[...]
            s16 = jnp.where(iota16 == 2 * k, bcl(vk, 0), s16)
            s16 = jnp.where(iota16 == 2 * k + 1, bcl(vk, 8), s16)
            f16 = jnp.where(iota16 == 2 * k, bcl(vk, 1), f16)
            f16 = jnp.where(iota16 == 2 * k + 1, bcl(vk, 9), f16)

        mgv = _allmax_f(s16)
        jgfv = _allmin_f(jnp.where(s16 == mgv, f16, f32(3.0e9)))
        jgv = jgfv.astype(i32)
        wtv = _allmin_i(jnp.where(f16 == jgfv, iota16, i32(99)))
        wt = wtv[0]

        @pl.when(t == wt)
        def _():
            rec[...] = recv
            pltpu.sync_copy(rec, shared.at[pl.ds(off + 128, 16)])

        plsc.subcore_barrier()
        pltpu.sync_copy(shared.at[pl.ds(off + 128, 16)], rec)
        w = rec[...]

        wx1 = bcl(w, 1)
        wy1 = bcl(w, 2)
        wx2 = bcl(w, 3)
        wy2 = bcl(w, 4)
        wa = bcl(w, 5)

        wm = bcl(w, 0)
        wc = bcl(w, 6)
        ws = jnp.take_along_axis(w, jnp.maximum(iota16 - 1, 0), axis=0)
        offv = wc * f32(_MAX_EDGE)
        out = jnp.where(iota16 == 0, wc,
              jnp.where(iota16 == 1, wm,
              jnp.where(iota16 <= 5, ws - offv, -1.0)))
        valid = jnp.minimum(jnp.maximum(wm * f32(1e30), 0.0), 1.0)
        out = out * valid - (1.0 - valid)
        outbuf[pl.ds(i * 16, 16)] = out

        return scan_pass(wx1, wy1, wx2, wy2, wa, jgv)

    lax.fori_loop(0, _NUM_PREDS, nms_iter, (bv0, bi0))

    @pl.when((t == 0) & (core == 0))
    def _():
        pltpu.sync_copy(outbuf, pred_hbm)


@jax.jit
def kernel(anchors, deltas, logits):
    f32 = jnp.float32
    pad = _NPAD - _N_BOXES

    def col(a, k):
        return jnp.pad(a[:, k], (0, pad)).reshape(_ROWS, _LANES)

    ax1, ay1, ax2, ay2 = (col(anchors, k) for k in range(4))
    dx, dy, dw, dh = (col(deltas, k) for k in range(4))

    sc8 = pl.pallas_call(
        _dense_body,
        out_shape=jax.ShapeDtypeStruct((8, _ROWS, _LANES), f32),
        in_specs=[pl.BlockSpec(memory_space=pltpu.VMEM)] * 9,
        out_specs=pl.BlockSpec(memory_space=pltpu.VMEM),
    )(logits, ax1, ay1, ax2, ay2, dx, dy, dw, dh)

    sc8r = sc8.reshape(8, _SC_TILES, _SC_CHUNK)

    mesh = plsc.VectorSubcoreMesh(
        core_axis_name="c", subcore_axis_name="s", num_cores=1)
    nms = pl.kernel(
        _nms_body,
        out_type=jax.ShapeDtypeStruct((_NUM_PREDS * 16,), f32),
        mesh=mesh,
        scratch_types=[
            pltpu.VMEM((8 * _SC_CHUNK,), f32),
            pltpu.VMEM((16,), f32),
            pltpu.VMEM((16,), f32),
            pltpu.VMEM((_NUM_PREDS * 16,), f32),
            pltpu.VMEM_SHARED((384,), f32),
        ],
    )
    pred = nms(sc8r)

    return pred.reshape(_NUM_PREDS, 16)[:, :6]


# speedup vs baseline: 2.3026x; 2.3026x over previous
import functools
import math

import jax
import jax.numpy as jnp
from jax import lax
from jax.experimental import pallas as pl
from jax.experimental.pallas import tpu as pltpu
from jax.experimental.pallas import tpu_sc as plsc

_N_BOXES = 20000
_NUM_CLASSES = 80
_NUM_PREDS = 100
_IOU_THR = 0.5
_SCORE_THR = 0.3
_MAX_EDGE = 1024
_SCALE_CLAMP = math.log(1000.0 / 16)

_ROWS = 160
_LANES = 128
_NPAD = _ROWS * _LANES

_SC_TILES = 16
_SC_CHUNK = _NPAD // _SC_TILES
_SC_STEPS = _SC_CHUNK // 16


def _dense_body(l_ref, ax1_ref, ay1_ref, ax2_ref, ay2_ref,
                dx_ref, dy_ref, dw_ref, dh_ref, sc8_ref):
    f32 = jnp.float32

    ax1, ay1, ax2, ay2 = ax1_ref[...], ay1_ref[...], ax2_ref[...], ay2_ref[...]
    dxv, dyv, dwv, dhv = dx_ref[...], dy_ref[...], dw_ref[...], dh_ref[...]
    widths = ax2 - ax1
    heights = ay2 - ay1
    ctr_x = (ax1 + ax2) * 0.5
    ctr_y = (ay1 + ay2) * 0.5
    dw = jnp.minimum(dwv, _SCALE_CLAMP)
    dh = jnp.minimum(dhv, _SCALE_CLAMP)
    pred_ctr_x = dxv * widths + ctr_x
    pred_ctr_y = dyv * heights + ctr_y
    pred_w = jnp.exp(dw) * widths
    pred_h = jnp.exp(dh) * heights
    hi = f32(_MAX_EDGE - 1.0)
    x1 = jnp.clip(pred_ctr_x - 0.5 * pred_w, 0.0, hi)
    y1 = jnp.clip(pred_ctr_y - 0.5 * pred_h, 0.0, hi)
    x2 = jnp.clip(pred_ctr_x + 0.5 * pred_w, 0.0, hi)
    y2 = jnp.clip(pred_ctr_y + 0.5 * pred_h, 0.0, hi)

    l = jnp.concatenate(
        [l_ref[...],
         jnp.full((_NPAD - _N_BOXES, _NUM_CLASSES), -100.0, jnp.float32)],
        axis=0).reshape(_ROWS, _LANES, _NUM_CLASSES)
    p = jax.nn.sigmoid(l)
    m = jnp.max(p, axis=2)
    cat = jnp.argmax(p, axis=2)
    catf = cat.astype(f32)
    s0 = jnp.where(m >= _SCORE_THR, m, -1.0)

    off = catf * f32(_MAX_EDGE)
    x1c = x1 + off
    y1c = y1 + off
    x2c = x2 + off
    y2c = y2 + off
    area = (x2c - x1c) * (y2c - y1c)

    sc8_ref[0] = s0
    sc8_ref[1] = x1c
    sc8_ref[2] = y1c
    sc8_ref[3] = x2c
    sc8_ref[4] = y2c
    sc8_ref[5] = area
    sc8_ref[6] = catf
    sc8_ref[7] = jnp.zeros((_ROWS, _LANES), f32)


def _allmax_f(v):
    iota = lax.iota(jnp.int32, 16)
    for k in (1, 2, 4, 8):
        v = jnp.maximum(v, jnp.take_along_axis(v, iota ^ k, axis=0))
    return v


def _allmin_f(v):
    iota = lax.iota(jnp.int32, 16)
    for k in (1, 2, 4, 8):
        v = jnp.minimum(v, jnp.take_along_axis(v, iota ^ k, axis=0))
    return v


def _allmin_i(v):
    iota = lax.iota(jnp.int32, 16)
    for k in (1, 2, 4, 8):
        v = jnp.minimum(v, jnp.take_along_axis(v, iota ^ k, axis=0))
    return v


def _nms_body(sc8_hbm, pred_hbm, loc, slot, rec, buf128, outbuf, shared):
    f32 = jnp.float32
    i32 = jnp.int32
    t = lax.axis_index("s")
    core = lax.axis_index("c")
    base = t * _SC_CHUNK
    iota16 = lax.iota(i32, 16)

    def bcl(v, k):
        return jnp.take_along_axis(v, iota16 * 0 + k, axis=0)

    for q in range(8):
        pltpu.sync_copy(sc8_hbm.at[q, t], loc.at[pl.ds(q * _SC_CHUNK, _SC_CHUNK)])

    def scan_pass(wx1, wy1, wx2, wy2, wa, jgv):
        def step(kk, carry):
            bv, bi = carry
            for u in range(8):
                o = kk * 128 + u * 16
                sv = loc[pl.ds(o, 16)]
                xx1 = jnp.maximum(wx1, loc[pl.ds(1 * _SC_CHUNK + o, 16)])
                yy1 = jnp.maximum(wy1, loc[pl.ds(2 * _SC_CHUNK + o, 16)])
                xx2 = jnp.minimum(wx2, loc[pl.ds(3 * _SC_CHUNK + o, 16)])
                yy2 = jnp.minimum(wy2, loc[pl.ds(4 * _SC_CHUNK + o, 16)])
                inter = jnp.maximum(xx2 - xx1, 0.0) * jnp.maximum(yy2 - yy1, 0.0)
                iou = inter / (wa + loc[pl.ds(5 * _SC_CHUNK + o, 16)] - inter + 1e-9)
                gidx = iota16 + (base + o)
                supp = (iou > _IOU_THR) | (gidx == jgv)
                snew = jnp.where(supp, -1.0, sv)
                loc[pl.ds(o, 16)] = snew
                cond = snew > bv
                bv = jnp.where(cond, snew, bv)
                bi = jnp.where(cond, gidx, bi)
            return bv, bi
        init = (jnp.full((16,), -2.0, f32), jnp.zeros((16,), i32))
        return lax.fori_loop(0, _SC_STEPS // 8, step, init)

    big = f32(3.0e38)
    bv0, bi0 = scan_pass(jnp.full((16,), big, f32), jnp.full((16,), big, f32),
                         jnp.full((16,), -big, f32), jnp.full((16,), -big, f32),
                         jnp.full((16,), 1.0, f32),
                         jnp.full((16,), -1, i32))

    def nms_iter(i, carry):
        bv, bi = carry
        mv = _allmax_f(bv)
        fidv = _allmin_i(jnp.where(bv == mv, bi, i32(2**30)))
        fid = fidv[0]
        jl = fid - base
        jb = (jl // 16) * 16
        ofsv = iota16 * 0 + (jl - jb)

        recv = jnp.full((16,), -1.0, f32)
        for q in range(7):
            wq = loc[pl.ds(q * _SC_CHUNK + jb, 16)]
            vq = jnp.take_along_axis(wq, ofsv, axis=0)
            recv = jnp.where(iota16 == q, vq, recv)
        recv = jnp.where(iota16 == 7, fidv.astype(f32), recv)

        off = (i % 2) * 192
        slot[...] = jnp.where(iota16 == 0, mv,
                    jnp.where(iota16 == 1, fidv.astype(f32), 0.0))
        pltpu.sync_copy(slot.at[pl.ds(0, 8)], shared.at[pl.ds(off + 8 * t, 8)])
        plsc.subcore_barrier()

        pltpu.sync_copy(shared.at[pl.ds(off, 128)], buf128)
        s16 = jnp.full((16,), -3.0, f32)
        f16 = jnp.full((16,), 0.0, f32)
        for k in range(8):
            vk = buf128[pl.ds(16 * k, 16)]
            s16 = jnp.where(iota16 == 2 * k, bcl(vk, 0), s16)
            s16 = jnp.where(iota16 == 2 * k + 1, bcl(vk, 8), s16)
            f16 = jnp.where(iota16 == 2 * k, bcl(vk, 1), f16)
            f16 = jnp.where(iota16 == 2 * k + 1, bcl(vk, 9), f16)

        mgv = _allmax_f(s16)
        jgfv = _allmin_f(jnp.where(s16 == mgv, f16, f32(3.0e9)))
        jgv = jgfv.astype(i32)
        wtv = _allmin_i(jnp.where(f16 == jgfv, iota16, i32(99)))
        wt = wtv[0]

        @pl.when(t == wt)
        def _():
            rec[...] = recv
            pltpu.sync_copy(rec, shared.at[pl.ds(off + 128, 16)])

        plsc.subcore_barrier()
        pltpu.sync_copy(shared.at[pl.ds(off + 128, 16)], rec)
        w = rec[...]

        wx1 = bcl(w, 1)
        wy1 = bcl(w, 2)
        wx2 = bcl(w, 3)
        wy2 = bcl(w, 4)
        wa = bcl(w, 5)

        wm = bcl(w, 0)
        wc = bcl(w, 6)
        ws = jnp.take_along_axis(w, jnp.maximum(iota16 - 1, 0), axis=0)
        offv = wc * f32(_MAX_EDGE)
        out = jnp.where(iota16 == 0, wc,
              jnp.where(iota16 == 1, wm,
              jnp.where(iota16 <= 5, ws - offv, -1.0)))
        valid = jnp.minimum(jnp.maximum(wm * f32(1e30), 0.0), 1.0)
        out = out * valid - (1.0 - valid)
        outbuf[pl.ds(i * 16, 16)] = out

        return scan_pass(wx1, wy1, wx2, wy2, wa, jgv)

    lax.fori_loop(0, _NUM_PREDS, nms_iter, (bv0, bi0))

    @pl.when((t == 0) & (core == 0))
    def _():
        pltpu.sync_copy(outbuf, pred_hbm)


@jax.jit
def kernel(anchors, deltas, logits):
    f32 = jnp.float32
    pad = _NPAD - _N_BOXES

    def col(a, k):
        return jnp.pad(a[:, k], (0, pad)).reshape(_ROWS, _LANES)

    ax1, ay1, ax2, ay2 = (col(anchors, k) for k in range(4))
    dx, dy, dw, dh = (col(deltas, k) for k in range(4))

    sc8 = pl.pallas_call(
        _dense_body,
        out_shape=jax.ShapeDtypeStruct((8, _ROWS, _LANES), f32),
        in_specs=[pl.BlockSpec(memory_space=pltpu.VMEM)] * 9,
        out_specs=pl.BlockSpec(memory_space=pltpu.VMEM),
    )(logits, ax1, ay1, ax2, ay2, dx, dy, dw, dh)

    sc8r = sc8.reshape(8, _SC_TILES, _SC_CHUNK)

    mesh = plsc.VectorSubcoreMesh(
        core_axis_name="c", subcore_axis_name="s", num_cores=1)
    nms = pl.kernel(
        _nms_body,
        out_type=jax.ShapeDtypeStruct((_NUM_PREDS * 16,), f32),
        mesh=mesh,
        scratch_types=[
            pltpu.VMEM((8 * _SC_CHUNK,), f32),
            pltpu.VMEM((16,), f32),
            pltpu.VMEM((16,), f32),
            pltpu.VMEM((128,), f32),
            pltpu.VMEM((_NUM_PREDS * 16,), f32),
            pltpu.VMEM_SHARED((384,), f32),
        ],
    )
    pred = nms(sc8r)

    return pred.reshape(_NUM_PREDS, 16)[:, :6]


# speedup vs baseline: 2.6406x; 1.1468x over previous
import functools
import math

import jax
import jax.numpy as jnp
from jax import lax
from jax.experimental import pallas as pl
from jax.experimental.pallas import tpu as pltpu
from jax.experimental.pallas import tpu_sc as plsc

_N_BOXES = 20000
_NUM_CLASSES = 80
_NUM_PREDS = 100
_IOU_THR = 0.5
_SCORE_THR = 0.3
_MAX_EDGE = 1024
_SCALE_CLAMP = math.log(1000.0 / 16)

_ROWS = 160
_LANES = 128
_NPAD = _ROWS * _LANES

_SC_TILES = 16
_SC_CHUNK = _NPAD // _SC_TILES
_SC_STEPS = _SC_CHUNK // 16


def _dense_body(l_ref, ax1_ref, ay1_ref, ax2_ref, ay2_ref,
                dx_ref, dy_ref, dw_ref, dh_ref, sc8_ref):
    f32 = jnp.float32

    ax1, ay1, ax2, ay2 = ax1_ref[...], ay1_ref[...], ax2_ref[...], ay2_ref[...]
    dxv, dyv, dwv, dhv = dx_ref[...], dy_ref[...], dw_ref[...], dh_ref[...]
    widths = ax2 - ax1
    heights = ay2 - ay1
    ctr_x = (ax1 + ax2) * 0.5
    ctr_y = (ay1 + ay2) * 0.5
    dw = jnp.minimum(dwv, _SCALE_CLAMP)
    dh = jnp.minimum(dhv, _SCALE_CLAMP)
    pred_ctr_x = dxv * widths + ctr_x
    pred_ctr_y = dyv * heights + ctr_y
    pred_w = jnp.exp(dw) * widths
    pred_h = jnp.exp(dh) * heights
    hi = f32(_MAX_EDGE - 1.0)
    x1 = jnp.clip(pred_ctr_x - 0.5 * pred_w, 0.0, hi)
    y1 = jnp.clip(pred_ctr_y - 0.5 * pred_h, 0.0, hi)
    x2 = jnp.clip(pred_ctr_x + 0.5 * pred_w, 0.0, hi)
    y2 = jnp.clip(pred_ctr_y + 0.5 * pred_h, 0.0, hi)

    l = jnp.concatenate(
        [l_ref[...],
         jnp.full((_NPAD - _N_BOXES, _NUM_CLASSES), -100.0, jnp.float32)],
        axis=0).reshape(_ROWS, _LANES, _NUM_CLASSES)
    p = jax.nn.sigmoid(l)
    m = jnp.max(p, axis=2)
    cat = jnp.argmax(p, axis=2)
    catf = cat.astype(f32)
    s0 = jnp.where(m >= _SCORE_THR, m, -1.0)

    off = catf * f32(_MAX_EDGE)
    x1c = x1 + off
    y1c = y1 + off
    x2c = x2 + off
    y2c = y2 + off
    area = (x2c - x1c) * (y2c - y1c)

    sc8_ref[0] = s0
    sc8_ref[1] = x1c
    sc8_ref[2] = y1c
    sc8_ref[3] = x2c
    sc8_ref[4] = y2c
    sc8_ref[5] = area
    sc8_ref[6] = catf
    sc8_ref[7] = jnp.zeros((_ROWS, _LANES), f32)


def _allmax_f(v):
    iota = lax.iota(jnp.int32, 16)
    for k in (1, 2, 4, 8):
        v = jnp.maximum(v, jnp.take_along_axis(v, iota ^ k, axis=0))
    return v


def _allmin_f(v):
    iota = lax.iota(jnp.int32, 16)
    for k in (1, 2, 4, 8):
        v = jnp.minimum(v, jnp.take_along_axis(v, iota ^ k, axis=0))
    return v


def _allmin_i(v):
    iota = lax.iota(jnp.int32, 16)
    for k in (1, 2, 4, 8):
        v = jnp.minimum(v, jnp.take_along_axis(v, iota ^ k, axis=0))
    return v


def _nms_body(sc8_hbm, pred_hbm, loc, slot, rec, buf128, outbuf, shared):
    f32 = jnp.float32
    i32 = jnp.int32
    t = lax.axis_index("s")
    core = lax.axis_index("c")
    base = t * _SC_CHUNK
    iota16 = lax.iota(i32, 16)

    def bcl(v, k):
        return jnp.take_along_axis(v, iota16 * 0 + k, axis=0)

    for q in range(8):
        pltpu.sync_copy(sc8_hbm.at[q, t], loc.at[pl.ds(q * _SC_CHUNK, _SC_CHUNK)])

    def scan_pass(wx1, wy1, wx2, wy2, wa, jgv):
        def step(kk, carry):
            bv, bi = carry
            for u in range(8):
                o = kk * 128 + u * 16
                sv = loc[pl.ds(o, 16)]
                xx1 = jnp.maximum(wx1, loc[pl.ds(1 * _SC_CHUNK + o, 16)])
                yy1 = jnp.maximum(wy1, loc[pl.ds(2 * _SC_CHUNK + o, 16)])
                xx2 = jnp.minimum(wx2, loc[pl.ds(3 * _SC_CHUNK + o, 16)])
                yy2 = jnp.minimum(wy2, loc[pl.ds(4 * _SC_CHUNK + o, 16)])
                inter = jnp.maximum(xx2 - xx1, 0.0) * jnp.maximum(yy2 - yy1, 0.0)
                iou = inter / (wa + loc[pl.ds(5 * _SC_CHUNK + o, 16)] - inter + 1e-9)
                gidx = iota16 + (base + o)
                supp = (iou > _IOU_THR) | (gidx == jgv)
                snew = jnp.where(supp, -1.0, sv)
                loc[pl.ds(o, 16)] = snew
                cond = snew > bv
                bv = jnp.where(cond, snew, bv)
                bi = jnp.where(cond, gidx, bi)
            return bv, bi
        init = (jnp.full((16,), -2.0, f32), jnp.zeros((16,), i32))
        return lax.fori_loop(0, _SC_STEPS // 8, step, init)

    big = f32(3.0e38)
    bv0, bi0 = scan_pass(jnp.full((16,), big, f32), jnp.full((16,), big, f32),
                         jnp.full((16,), -big, f32), jnp.full((16,), -big, f32),
                         jnp.full((16,), 1.0, f32),
                         jnp.full((16,), -1, i32))

    def nms_iter(i, carry):
        bv, bi = carry
        mv = _allmax_f(bv)
        fidv = _allmin_i(jnp.where(bv == mv, bi, i32(2**30)))
        fid = fidv[0]
        jl = fid - base
        jb = (jl // 16) * 16
        ofsv = iota16 * 0 + (jl - jb)

        recv = jnp.full((16,), -1.0, f32)
        for q in range(7):
            wq = loc[pl.ds(q * _SC_CHUNK + jb, 16)]
            vq = jnp.take_along_axis(wq, ofsv, axis=0)
            recv = jnp.where(iota16 == q, vq, recv)
        recv = jnp.where(iota16 == 7, fidv.astype(f32), recv)

        off = (i % 2) * 256
        rec[...] = recv
        pltpu.sync_copy(rec, shared.at[pl.ds(off + 16 * t, 16)])
        plsc.subcore_barrier()

        pltpu.sync_copy(shared.at[pl.ds(off, 256)], buf128)
        s16 = jnp.full((16,), -3.0, f32)
        f16 = jnp.full((16,), 0.0, f32)
        for k in range(16):
            vk = buf128[pl.ds(16 * k, 16)]
            s16 = jnp.where(iota16 == k, bcl(vk, 0), s16)
            f16 = jnp.where(iota16 == k, bcl(vk, 7), f16)

        mgv = _allmax_f(s16)
        jgfv = _allmin_f(jnp.where(s16 == mgv, f16, f32(3.0e9)))
        jgv = jgfv.astype(i32)
        wtv = _allmin_i(jnp.where(f16 == jgfv, iota16, i32(99)))
        wt = wtv[0]
        w = buf128[pl.ds(wt * 16, 16)]

        wx1 = bcl(w, 1)
        wy1 = bcl(w, 2)
        wx2 = bcl(w, 3)
        wy2 = bcl(w, 4)
        wa = bcl(w, 5)

        wm = bcl(w, 0)
        wc = bcl(w, 6)
        ws = jnp.take_along_axis(w, jnp.maximum(iota16 - 1, 0), axis=0)
        offv = wc * f32(_MAX_EDGE)
        out = jnp.where(iota16 == 0, wc,
              jnp.where(iota16 == 1, wm,
              jnp.where(iota16 <= 5, ws - offv, -1.0)))
        valid = jnp.minimum(jnp.maximum(wm * f32(1e30), 0.0), 1.0)
        out = out * valid - (1.0 - valid)
        outbuf[pl.ds(i * 16, 16)] = out

        return scan_pass(wx1, wy1, wx2, wy2, wa, jgv)

    lax.fori_loop(0, _NUM_PREDS, nms_iter, (bv0, bi0))

    @pl.when((t == 0) & (core == 0))
    def _():
        pltpu.sync_copy(outbuf, pred_hbm)


@jax.jit
def kernel(anchors, deltas, logits):
    f32 = jnp.float32
    pad = _NPAD - _N_BOXES

    def col(a, k):
        return jnp.pad(a[:, k], (0, pad)).reshape(_ROWS, _LANES)

    ax1, ay1, ax2, ay2 = (col(anchors, k) for k in range(4))
    dx, dy, dw, dh = (col(deltas, k) for k in range(4))

    sc8 = pl.pallas_call(
        _dense_body,
        out_shape=jax.ShapeDtypeStruct((8, _ROWS, _LANES), f32),
        in_specs=[pl.BlockSpec(memory_space=pltpu.VMEM)] * 9,
        out_specs=pl.BlockSpec(memory_space=pltpu.VMEM),
    )(logits, ax1, ay1, ax2, ay2, dx, dy, dw, dh)

    sc8r = sc8.reshape(8, _SC_TILES, _SC_CHUNK)

    mesh = plsc.VectorSubcoreMesh(
        core_axis_name="c", subcore_axis_name="s", num_cores=1)
    nms = pl.kernel(
        _nms_body,
        out_type=jax.ShapeDtypeStruct((_NUM_PREDS * 16,), f32),
        mesh=mesh,
        scratch_types=[
            pltpu.VMEM((8 * _SC_CHUNK,), f32),
            pltpu.VMEM((16,), f32),
            pltpu.VMEM((16,), f32),
            pltpu.VMEM((256,), f32),
            pltpu.VMEM((_NUM_PREDS * 16,), f32),
            pltpu.VMEM_SHARED((512,), f32),
        ],
    )
    pred = nms(sc8r)

    return pred.reshape(_NUM_PREDS, 16)[:, :6]
